# baseline (speedup 1.0000x reference)
import jax
import jax.numpy as jnp
from jax import lax
from jax.experimental import pallas as pl
from jax.experimental.pallas import tpu as pltpu
from jax.experimental.pallas import tpu_sc as plsc

_B = 16384
_N = 64
_MARGIN = 1.0
_NW = 32
_RPW = _B // _NW
_L = 16
_W = 128
_PRW = _RPW * _N // _W
_NSLAB = 2
_PSLAB = _PRW // _NSLAB


def _sc_body(scores_hbm, wrT_hbm, out_hbm, chunk, wr0, partial, sems, semw):
    cid = lax.axis_index("c")
    sid = lax.axis_index("s")
    wid = sid * 2 + cid
    iota = lax.iota(jnp.int32, _L)

    copies = []
    for k in range(_NSLAB):
        copies.append(pltpu.async_copy(
            scores_hbm.at[pl.ds(wid * _PRW + k * _PSLAB, _PSLAB), :],
            chunk.at[pl.ds(k * _PSLAB, _PSLAB), :], sems[k]))
    wr_copy = pltpu.async_copy(
        wrT_hbm.at[pl.ds(0, 8), pl.ds(wid * _RPW, _RPW)], wr0, semw)

    accs = (jnp.zeros((_L,), jnp.float32),) * 4

    def row_body(q, accs):
        qs = jnp.full((_L,), q, jnp.int32)
        new = list(accs)
        for h in range(2):
            posplat = plsc.load_gather(
                chunk, [qs, jnp.full((_L,), h * _N, jnp.int32)])
            mb = posplat - jnp.float32(_MARGIN)
            for j in range(_N // _L):
                v = chunk[q, pl.ds(h * _N + j * _L, _L)]
                new[j] = new[j] + jnp.maximum(v, mb)
        return tuple(new)

    for k in range(_NSLAB):
        copies[k].wait()
        accs = lax.fori_loop(k * _PSLAB, (k + 1) * _PSLAB, row_body, accs,
                             unroll=4)

    wr_copy.wait()
    racc = jnp.zeros((_L,), jnp.float32)
    msum = jnp.zeros((_L,), jnp.float32)
    for m in range(_RPW // _L):
        rows = m * _L + iota
        q = lax.shift_right_logical(rows, 1)
        hcol = (rows & 1) * _N
        r0 = wr0[0, pl.ds(m * _L, _L)]
        posv = plsc.load_gather(chunk, [q, hcol])
        g = plsc.load_gather(chunk, [q, hcol + r0])
        mb = posv - jnp.float32(_MARGIN)
        racc = racc + jnp.maximum(g, mb)
        msum = msum + mb

    total = accs[0] + accs[1] + accs[2] + accs[3] - racc
    partial[...] = total * jnp.float32(1.0 / (_N - 1)) - msum
    pltpu.sync_copy(partial, out_hbm.at[wid])


def kernel(scores, nBestIndex, werRank):
    s2d = scores.reshape(_B * _N // _W, _W)
    wrT = werRank.T
    mesh = plsc.VectorSubcoreMesh(core_axis_name="c", subcore_axis_name="s")
    out = pl.kernel(
        _sc_body,
        mesh=mesh,
        out_type=jax.ShapeDtypeStruct((_NW, _L), jnp.float32),
        scratch_types=[
            pltpu.VMEM((_PRW, _W), jnp.float32),
            pltpu.VMEM((8, _RPW), jnp.int32),
            pltpu.VMEM((_L,), jnp.float32),
            [pltpu.SemaphoreType.DMA] * _NSLAB,
            pltpu.SemaphoreType.DMA,
        ],
        compiler_params=pltpu.CompilerParams(needs_layout_passes=False),
    )(s2d, wrT)
    return jnp.sum(out).reshape(1)


# speedup vs baseline: 1.0054x; 1.0054x over previous
import jax
import jax.numpy as jnp
from jax import lax
from jax.experimental import pallas as pl
from jax.experimental.pallas import tpu as pltpu
from jax.experimental.pallas import tpu_sc as plsc

_B = 16384
_N = 64
_MARGIN = 1.0
_NW = 32
_RPW = _B // _NW
_L = 16
_W = 128
_PRW = _RPW * _N // _W
_NSLAB = 2
_PSLAB = _PRW // _NSLAB


def _sc_body(scores_hbm, wrT_hbm, out_hbm, chunk, wr0, partial, sems, semw):
    cid = lax.axis_index("c")
    sid = lax.axis_index("s")
    wid = sid * 2 + cid
    iota = lax.iota(jnp.int32, _L)

    copies = []
    for k in range(_NSLAB):
        copies.append(pltpu.async_copy(
            scores_hbm.at[pl.ds(wid * _PRW + k * _PSLAB, _PSLAB), :],
            chunk.at[pl.ds(k * _PSLAB, _PSLAB), :], sems[k]))
    wr_copy = pltpu.async_copy(
        wrT_hbm.at[pl.ds(0, 8), pl.ds(wid * _RPW, _RPW)], wr0, semw)

    accs = (jnp.zeros((_L,), jnp.float32),) * 4

    def row_body(q, accs):
        qs = jnp.full((_L,), q, jnp.int32)
        new = list(accs)
        for h in range(2):
            posplat = plsc.load_gather(
                chunk, [qs, jnp.full((_L,), h * _N, jnp.int32)])
            mb = posplat - jnp.float32(_MARGIN)
            for j in range(_N // _L):
                v = chunk[q, pl.ds(h * _N + j * _L, _L)]
                new[j] = new[j] + jnp.maximum(v, mb)
        return tuple(new)

    for k in range(_NSLAB):
        copies[k].wait()
        accs = lax.fori_loop(k * _PSLAB, (k + 1) * _PSLAB, row_body, accs,
                             unroll=2)

    wr_copy.wait()
    racc = jnp.zeros((_L,), jnp.float32)
    msum = jnp.zeros((_L,), jnp.float32)
    for m in range(_RPW // _L):
        rows = m * _L + iota
        q = lax.shift_right_logical(rows, 1)
        hcol = (rows & 1) * _N
        r0 = wr0[0, pl.ds(m * _L, _L)]
        posv = plsc.load_gather(chunk, [q, hcol])
        g = plsc.load_gather(chunk, [q, hcol + r0])
        mb = posv - jnp.float32(_MARGIN)
        racc = racc + jnp.maximum(g, mb)
        msum = msum + mb

    total = accs[0] + accs[1] + accs[2] + accs[3] - racc
    partial[...] = total * jnp.float32(1.0 / (_N - 1)) - msum
    pltpu.sync_copy(partial, out_hbm.at[wid])


def kernel(scores, nBestIndex, werRank):
    s2d = scores.reshape(_B * _N // _W, _W)
    wrT = werRank.T
    mesh = plsc.VectorSubcoreMesh(core_axis_name="c", subcore_axis_name="s")
    out = pl.kernel(
        _sc_body,
        mesh=mesh,
        out_type=jax.ShapeDtypeStruct((_NW, _L), jnp.float32),
        scratch_types=[
            pltpu.VMEM((_PRW, _W), jnp.float32),
            pltpu.VMEM((8, _RPW), jnp.int32),
            pltpu.VMEM((_L,), jnp.float32),
            [pltpu.SemaphoreType.DMA] * _NSLAB,
            pltpu.SemaphoreType.DMA,
        ],
        compiler_params=pltpu.CompilerParams(needs_layout_passes=False),
    )(s2d, wrT)
    return jnp.sum(out).reshape(1)
